# baseline (speedup 1.0000x reference)
import functools

import jax
import jax.numpy as jnp
from jax.experimental import pallas as pl
from jax.experimental.pallas import tpu as pltpu

_NUM_GENES = 1000
_NUM_HIDDEN = 64
_TILE_M = 512


def _fused_kernel(x_ref, emb_ref, scale_ref, bias_ref, out_ref):
    x = x_ref[...]
    emb = emb_ref[...]
    out = jax.lax.dot_general(
        x, emb, (((1,), (0,)), ((), ())), preferred_element_type=jnp.float32
    )
    out = 0.5 * out * (1.0 + jax.lax.erf(out * (2.0 ** -0.5)))
    mu = jnp.mean(out, axis=-1, keepdims=True)
    var = jnp.mean((out - mu) ** 2, axis=-1, keepdims=True)
    out = (out - mu) / jnp.sqrt(var + 1e-5) * scale_ref[...] + bias_ref[...]
    out_ref[...] = out


@functools.partial(jax.jit, static_argnames=())
def kernel(x, emb, ln_scale, ln_bias):
    batch, num_genes = x.shape
    num_hidden = emb.shape[1]
    grid = (batch // _TILE_M,)
    out = pl.pallas_call(
        _fused_kernel,
        grid=grid,
        in_specs=[
            pl.BlockSpec((_TILE_M, num_genes), lambda i: (i, 0)),
            pl.BlockSpec((num_genes, num_hidden), lambda i: (0, 0)),
            pl.BlockSpec((1, num_hidden), lambda i: (0, 0)),
            pl.BlockSpec((1, num_hidden), lambda i: (0, 0)),
        ],
        out_specs=pl.BlockSpec((_TILE_M, num_hidden), lambda i: (i, 0)),
        out_shape=jax.ShapeDtypeStruct((batch, num_hidden), jnp.float32),
        compiler_params=pltpu.CompilerParams(
            dimension_semantics=("arbitrary",),
        ),
    )(x, emb, ln_scale.reshape(1, num_hidden), ln_bias.reshape(1, num_hidden))
    gene_idx = jnp.arange(num_genes, dtype=jnp.int32)
    return (out, gene_idx)


# speedup vs baseline: 1.0411x; 1.0411x over previous
import functools

import jax
import jax.numpy as jnp
from jax.experimental import pallas as pl
from jax.experimental.pallas import tpu as pltpu

_NUM_GENES = 1000
_NUM_HIDDEN = 64
_TILE_M = 512


def _fused_kernel(x_ref, emb_ref, scale_ref, bias_ref, out_ref):
    x = x_ref[...]
    emb = emb_ref[...]
    out = jax.lax.dot_general(
        x, emb, (((1,), (0,)), ((), ())), preferred_element_type=jnp.float32
    )
    out = 0.5 * out * (1.0 + jax.lax.erf(out * (2.0 ** -0.5)))
    mu = jnp.mean(out, axis=-1, keepdims=True)
    var = jnp.mean((out - mu) ** 2, axis=-1, keepdims=True)
    out = (out - mu) / jnp.sqrt(var + 1e-5) * scale_ref[...] + bias_ref[...]
    out_ref[...] = out


@functools.partial(jax.jit, static_argnames=())
def kernel(x, emb, ln_scale, ln_bias):
    batch, num_genes = x.shape
    num_hidden = emb.shape[1]
    grid = (batch // _TILE_M,)
    out = pl.pallas_call(
        _fused_kernel,
        grid=grid,
        in_specs=[
            pl.BlockSpec((_TILE_M, num_genes), lambda i: (i, 0)),
            pl.BlockSpec((num_genes, num_hidden), lambda i: (0, 0)),
            pl.BlockSpec((1, num_hidden), lambda i: (0, 0)),
            pl.BlockSpec((1, num_hidden), lambda i: (0, 0)),
        ],
        out_specs=pl.BlockSpec((_TILE_M, num_hidden), lambda i: (i, 0)),
        out_shape=jax.ShapeDtypeStruct((batch, num_hidden), jnp.float32),
        compiler_params=pltpu.CompilerParams(
            dimension_semantics=("parallel",),
        ),
    )(x, emb, ln_scale.reshape(1, num_hidden), ln_bias.reshape(1, num_hidden))
    gene_idx = jnp.arange(num_genes, dtype=jnp.int32)
    return (out, gene_idx)


# speedup vs baseline: 1.1611x; 1.1153x over previous
import functools

import jax
import jax.numpy as jnp
from jax.experimental import pallas as pl
from jax.experimental.pallas import tpu as pltpu

_NUM_GENES = 1000
_NUM_HIDDEN = 64
_TILE_M = 512
_SPLIT = 4


def _fused_kernel(*refs):
    x_refs = refs[:_SPLIT]
    emb_ref, scale_ref, bias_ref, out_ref = refs[_SPLIT:]
    emb = emb_ref[...]
    scale = scale_ref[...]
    bias = bias_ref[...]
    for j in range(_SPLIT):
        x = x_refs[j][...]
        out = jax.lax.dot_general(
            x, emb, (((1,), (0,)), ((), ())), preferred_element_type=jnp.float32
        )
        out = 0.5 * out * (1.0 + jax.lax.erf(out * (2.0 ** -0.5)))
        mu = jnp.mean(out, axis=-1, keepdims=True)
        var = jnp.mean((out - mu) ** 2, axis=-1, keepdims=True)
        out = (out - mu) / jnp.sqrt(var + 1e-5) * scale + bias
        out_ref[pl.ds(j * _TILE_M, _TILE_M), :] = out


@functools.partial(jax.jit, static_argnames=())
def kernel(x, emb, ln_scale, ln_bias):
    batch, num_genes = x.shape
    num_hidden = emb.shape[1]
    rows_per_step = _TILE_M * _SPLIT
    grid = (batch // rows_per_step,)
    x_specs = [
        pl.BlockSpec((_TILE_M, num_genes), lambda i, j=j: (i * _SPLIT + j, 0))
        for j in range(_SPLIT)
    ]
    out = pl.pallas_call(
        _fused_kernel,
        grid=grid,
        in_specs=x_specs + [
            pl.BlockSpec((num_genes, num_hidden), lambda i: (0, 0)),
            pl.BlockSpec((1, num_hidden), lambda i: (0, 0)),
            pl.BlockSpec((1, num_hidden), lambda i: (0, 0)),
        ],
        out_specs=pl.BlockSpec((rows_per_step, num_hidden), lambda i: (i, 0)),
        out_shape=jax.ShapeDtypeStruct((batch, num_hidden), jnp.float32),
        compiler_params=pltpu.CompilerParams(
            dimension_semantics=("arbitrary",),
        ),
    )(*([x] * _SPLIT), emb,
      ln_scale.reshape(1, num_hidden), ln_bias.reshape(1, num_hidden))
    gene_idx = jnp.arange(num_genes, dtype=jnp.int32)
    return (out, gene_idx)


# speedup vs baseline: 3.6484x; 3.1422x over previous
import functools

import jax
import jax.numpy as jnp
from jax.experimental import pallas as pl
from jax.experimental.pallas import tpu as pltpu

_TILE_N = 1024


def _fused_kernel(xt_ref, embt_ref, scale_ref, bias_ref, out_ref):
    xt = xt_ref[...]
    embt = embt_ref[...]
    out = jax.lax.dot_general(
        embt, xt, (((1,), (0,)), ((), ())), preferred_element_type=jnp.float32
    )
    out = 0.5 * out * (1.0 + jax.lax.erf(out * (2.0 ** -0.5)))
    mu = jnp.mean(out, axis=0, keepdims=True)
    var = jnp.mean((out - mu) ** 2, axis=0, keepdims=True)
    out = (out - mu) / jnp.sqrt(var + 1e-5) * scale_ref[...] + bias_ref[...]
    out_ref[...] = out


@functools.partial(jax.jit, static_argnames=())
def kernel(x, emb, ln_scale, ln_bias):
    batch, num_genes = x.shape
    num_hidden = emb.shape[1]
    xt = x.T
    embt = emb.T
    grid = (batch // _TILE_N,)
    out_t = pl.pallas_call(
        _fused_kernel,
        grid=grid,
        in_specs=[
            pl.BlockSpec((num_genes, _TILE_N), lambda i: (0, i)),
            pl.BlockSpec((num_hidden, num_genes), lambda i: (0, 0)),
            pl.BlockSpec((num_hidden, 1), lambda i: (0, 0)),
            pl.BlockSpec((num_hidden, 1), lambda i: (0, 0)),
        ],
        out_specs=pl.BlockSpec((num_hidden, _TILE_N), lambda i: (0, i)),
        out_shape=jax.ShapeDtypeStruct((num_hidden, batch), jnp.float32),
        compiler_params=pltpu.CompilerParams(
            dimension_semantics=("arbitrary",),
        ),
    )(xt, embt, ln_scale.reshape(num_hidden, 1), ln_bias.reshape(num_hidden, 1))
    gene_idx = jnp.arange(num_genes, dtype=jnp.int32)
    return (out_t.T, gene_idx)


# speedup vs baseline: 4.0040x; 1.0975x over previous
import functools

import jax
import jax.numpy as jnp
from jax.experimental import pallas as pl
from jax.experimental.pallas import tpu as pltpu

_TILE_N = 1024


def _fused_kernel(xt_ref, embt_ref, scale_ref, bias_ref, out_ref):
    xt = xt_ref[...]
    embt = embt_ref[...]
    out = jax.lax.dot_general(
        embt, xt, (((1,), (0,)), ((), ())), preferred_element_type=jnp.float32
    )
    out = 0.5 * out * (1.0 + jax.lax.erf(out * (2.0 ** -0.5)))
    mu = jnp.mean(out, axis=0, keepdims=True)
    var = jnp.mean((out - mu) ** 2, axis=0, keepdims=True)
    scale = scale_ref[...].reshape(-1, 1)
    bias = bias_ref[...].reshape(-1, 1)
    out = (out - mu) / jnp.sqrt(var + 1e-5) * scale + bias
    out_ref[...] = out


@functools.partial(jax.jit, static_argnames=())
def kernel(x, emb, ln_scale, ln_bias):
    batch, num_genes = x.shape
    num_hidden = emb.shape[1]
    xt = x.T
    embt = emb.T
    grid = (batch // _TILE_N,)
    out_t = pl.pallas_call(
        _fused_kernel,
        grid=grid,
        in_specs=[
            pl.BlockSpec((num_genes, _TILE_N), lambda i: (0, i)),
            pl.BlockSpec((num_hidden, num_genes), lambda i: (0, 0)),
            pl.BlockSpec((1, num_hidden), lambda i: (0, 0)),
            pl.BlockSpec((1, num_hidden), lambda i: (0, 0)),
        ],
        out_specs=pl.BlockSpec((num_hidden, _TILE_N), lambda i: (0, i)),
        out_shape=jax.ShapeDtypeStruct((num_hidden, batch), jnp.float32),
        compiler_params=pltpu.CompilerParams(
            dimension_semantics=("arbitrary",),
        ),
    )(xt, embt, ln_scale.reshape(1, num_hidden), ln_bias.reshape(1, num_hidden))
    gene_idx = jnp.arange(num_genes, dtype=jnp.int32)
    return (out_t.T, gene_idx)


# speedup vs baseline: 4.4664x; 1.1155x over previous
import functools

import jax
import jax.numpy as jnp
from jax.experimental import pallas as pl
from jax.experimental.pallas import tpu as pltpu

_TILE_N = 2048


def _fused_kernel(xt_ref, embt_ref, scale_ref, bias_ref, out_ref):
    xt = xt_ref[...]
    embt = embt_ref[...]
    out = jax.lax.dot_general(
        embt, xt, (((1,), (0,)), ((), ())), preferred_element_type=jnp.float32
    )
    out = 0.5 * out * (1.0 + jax.lax.erf(out * (2.0 ** -0.5)))
    mu = jnp.mean(out, axis=0, keepdims=True)
    var = jnp.mean((out - mu) ** 2, axis=0, keepdims=True)
    scale = scale_ref[...].reshape(-1, 1)
    bias = bias_ref[...].reshape(-1, 1)
    out = (out - mu) / jnp.sqrt(var + 1e-5) * scale + bias
    out_ref[...] = out


@functools.partial(jax.jit, static_argnames=())
def kernel(x, emb, ln_scale, ln_bias):
    batch, num_genes = x.shape
    num_hidden = emb.shape[1]
    xt = x.T
    embt = emb.T
    grid = (batch // _TILE_N,)
    out_t = pl.pallas_call(
        _fused_kernel,
        grid=grid,
        in_specs=[
            pl.BlockSpec((num_genes, _TILE_N), lambda i: (0, i)),
            pl.BlockSpec((num_hidden, num_genes), lambda i: (0, 0)),
            pl.BlockSpec((1, num_hidden), lambda i: (0, 0)),
            pl.BlockSpec((1, num_hidden), lambda i: (0, 0)),
        ],
        out_specs=pl.BlockSpec((num_hidden, _TILE_N), lambda i: (0, i)),
        out_shape=jax.ShapeDtypeStruct((num_hidden, batch), jnp.float32),
        compiler_params=pltpu.CompilerParams(
            dimension_semantics=("arbitrary",),
        ),
    )(xt, embt, ln_scale.reshape(1, num_hidden), ln_bias.reshape(1, num_hidden))
    gene_idx = jnp.arange(num_genes, dtype=jnp.int32)
    return (out_t.T, gene_idx)
